# baseline (speedup 1.0000x reference)
import functools

import jax
import jax.numpy as jnp
from jax import lax
from jax.experimental import pallas as pl
from jax.experimental.pallas import tpu as pltpu
from jax.experimental.pallas import tpu_sc as plsc

_NC, _NS, _L = 2, 16, 16
_NW = _NC * _NS
_B = 16384
_D = 32
_V = 1000000
_SR = 128
_BPW = _B // _NW

_sc_mesh = plsc.VectorSubcoreMesh(core_axis_name="c", subcore_axis_name="s")


@functools.partial(
    pl.kernel,
    out_type=jax.ShapeDtypeStruct((_B, _SR), jnp.float32),
    mesh=_sc_mesh,
    scratch_types=[
        pltpu.VMEM((_BPW,), jnp.int32),
        pltpu.VMEM((_BPW, _SR), jnp.float32),
        pltpu.SemaphoreType.DMA,
    ],
)
def _sc_rowgather(idx_hbm, table_hbm, out_hbm, idx_v, rows_v, sem):
    wid = lax.axis_index("s") * _NC + lax.axis_index("c")
    base = wid * _BPW
    pltpu.sync_copy(idx_hbm.at[pl.ds(base, _BPW)], idx_v)
    pltpu.async_copy(table_hbm.at[idx_v], rows_v, sem).wait()
    pltpu.sync_copy(rows_v, out_hbm.at[pl.ds(base, _BPW)])


def _mlp_body(us_ref, ms_ref, ur_ref, mr_ref, w1u_ref, w1m_ref, b1_ref,
              w2_ref, b2_ref, w3_ref, b3_ref, out_ref):
    us = us_ref[...]
    ms = ms_ref[...]
    ur = ur_ref[...]
    mr = mr_ref[...]
    u = jnp.zeros((us.shape[0], _D), jnp.float32)
    m = jnp.zeros((us.shape[0], _D), jnp.float32)
    for r in range(4):
        u = u + jnp.where(ur == r, 1.0, 0.0) * us[:, r * _D:(r + 1) * _D]
        m = m + jnp.where(mr == r, 1.0, 0.0) * ms[:, r * _D:(r + 1) * _D]
    h = jnp.dot(u, w1u_ref[...], preferred_element_type=jnp.float32)
    h = h + jnp.dot(m, w1m_ref[...], preferred_element_type=jnp.float32)
    h = jnp.maximum(h + b1_ref[...], 0.0)
    h = jnp.dot(h, w2_ref[...], preferred_element_type=jnp.float32)
    h = jnp.maximum(h + b2_ref[...], 0.0)
    o = jnp.dot(h, w3_ref[...], preferred_element_type=jnp.float32)
    out_ref[...] = o + b3_ref[...]


def kernel(user, movie, user_emb, movie_emb, W1, b1, W2, b2, W3, b3):
    user = user.astype(jnp.int32)
    movie = movie.astype(jnp.int32)
    ue4 = user_emb.reshape(_V // 4, _SR)
    me4 = movie_emb.reshape(_V // 4, _SR)
    u_super = _sc_rowgather(jnp.right_shift(user, 2), ue4)
    m_super = _sc_rowgather(jnp.right_shift(movie, 2), me4)
    blk = 2048
    full = lambda s: pl.BlockSpec(s, lambda i: (0, 0))
    out = pl.pallas_call(
        _mlp_body,
        grid=(_B // blk,),
        in_specs=[
            pl.BlockSpec((blk, _SR), lambda i: (i, 0)),
            pl.BlockSpec((blk, _SR), lambda i: (i, 0)),
            pl.BlockSpec((blk, 1), lambda i: (i, 0)),
            pl.BlockSpec((blk, 1), lambda i: (i, 0)),
            full((_D, 64)), full((_D, 64)), full((1, 64)),
            full((64, 32)), full((1, 32)),
            full((32, 1)), full((1, 1)),
        ],
        out_specs=pl.BlockSpec((blk, 1), lambda i: (i, 0)),
        out_shape=jax.ShapeDtypeStruct((_B, 1), jnp.float32),
    )(u_super, m_super,
      jnp.bitwise_and(user, 3).reshape(_B, 1),
      jnp.bitwise_and(movie, 3).reshape(_B, 1),
      W1[:, :_D].T, W1[:, _D:].T, b1.reshape(1, 64),
      W2.T, b2.reshape(1, 32),
      W3.T, b3.reshape(1, 1))
    return out.reshape(_B)
